# baseline (speedup 1.0000x reference)
import functools

import jax
import jax.numpy as jnp
from jax import lax
from jax.experimental import pallas as pl
from jax.experimental.pallas import tpu as pltpu
from jax.experimental.pallas import tpu_sc as plsc

N = 10000
E = 320000
D = 128
DO = 16
NC = 2
NS = 16
NW = NC * NS
EB = 128
NBW = 80
NBLK = NW * NBW
E2 = NBLK * EB
NPAIR = NBW // 2
NP = 10080
ZR = 80
NZ = NP // ZR
L = 16
BN_EPS = 1e-5
RB = 1000

_mesh = plsc.VectorSubcoreMesh(core_axis_name="c", subcore_axis_name="s")
_sc_params = pltpu.CompilerParams(use_tc_tiling_on_sc=False)


@functools.partial(
    pl.kernel,
    out_type=jax.ShapeDtypeStruct((NC, NP, L), jnp.float32),
    mesh=_mesh,
    compiler_params=_sc_params,
    scratch_types=[
        pltpu.VMEM((EB, L), jnp.float32),
        pltpu.VMEM((1, EB), jnp.int32),
        pltpu.VMEM((1, EB), jnp.int32),
        pltpu.VMEM((ZR, L), jnp.float32),
        pltpu.VMEM_SHARED((NP, L), jnp.float32),
        pltpu.SemaphoreType.DMA,
        pltpu.SemaphoreType.DMA,
    ],
)
def _sc_count(dst_hbm, acc_hbm, ones_v, didx_a, didx_b, zbuf_v, shared,
              sem_a, sem_b):
    c = lax.axis_index("c")
    s = lax.axis_index("s")
    w = s * NC + c
    base = w * NBW

    @pl.loop(0, EB)
    def _(i):
        ones_v[i, :] = jnp.full((L,), 1.0, jnp.float32)

    @pl.loop(0, ZR)
    def _(i):
        zbuf_v[i, :] = jnp.zeros((L,), jnp.float32)

    @pl.loop(s, NZ, step=NS)
    def _(i):
        pltpu.sync_copy(zbuf_v, shared.at[pl.ds(i * ZR, ZR)])

    plsc.subcore_barrier()

    def start_idx(buf, sem, k):
        pltpu.async_copy(dst_hbm.at[pl.ds(base + k, 1)], buf, sem)

    def wait_idx(buf, sem):
        pltpu.make_async_copy(dst_hbm.at[pl.ds(base, 1)], buf, sem).wait()

    start_idx(didx_a, sem_a, 0)
    start_idx(didx_b, sem_b, 1)

    @pl.loop(0, NPAIR)
    def _(j):
        wait_idx(didx_a, sem_a)
        pltpu.sync_copy(ones_v, shared.at[didx_a.at[0]], add=True)

        @pl.when(j < NPAIR - 1)
        def _():
            start_idx(didx_a, sem_a, 2 * j + 2)

        wait_idx(didx_b, sem_b)
        pltpu.sync_copy(ones_v, shared.at[didx_b.at[0]], add=True)

        @pl.when(j < NPAIR - 1)
        def _():
            start_idx(didx_b, sem_b, 2 * j + 3)

    plsc.subcore_barrier()

    @pl.loop(s, NZ, step=NS)
    def _(i):
        pltpu.sync_copy(shared.at[pl.ds(i * ZR, ZR)],
                        acc_hbm.at[c].at[pl.ds(i * ZR, ZR)])


def _make_sc_propagate(width):
    @functools.partial(
        pl.kernel,
        out_type=jax.ShapeDtypeStruct((NC, NP, width), jnp.float32),
        mesh=_mesh,
        compiler_params=_sc_params,
        scratch_types=[
            pltpu.VMEM((1, EB), jnp.int32),
            pltpu.VMEM((1, EB), jnp.int32),
            pltpu.VMEM((1, EB), jnp.int32),
            pltpu.VMEM((1, EB), jnp.int32),
            pltpu.VMEM((EB, width), jnp.float32),
            pltpu.VMEM((EB, width), jnp.float32),
            pltpu.VMEM((ZR, width), jnp.float32),
            pltpu.VMEM_SHARED((NP, width), jnp.float32),
            pltpu.SemaphoreType.DMA,
            pltpu.SemaphoreType.DMA,
            pltpu.SemaphoreType.DMA,
            pltpu.SemaphoreType.DMA,
        ],
    )
    def _sc_prop(vals_hbm, src_hbm, dst_hbm, acc_hbm,
                 sidx_a, didx_a, sidx_b, didx_b, rows_a, rows_b,
                 zbuf_v, shared, sem_ia, sem_ib, sem_ga, sem_gb):
        c = lax.axis_index("c")
        s = lax.axis_index("s")
        w = s * NC + c
        base = w * NBW

        @pl.loop(0, ZR)
        def _(i):
            @pl.loop(0, width, step=L)
            def _(j):
                zbuf_v[i, pl.ds(j, L)] = jnp.zeros((L,), jnp.float32)

        @pl.loop(s, NZ, step=NS)
        def _(i):
            pltpu.sync_copy(zbuf_v, shared.at[pl.ds(i * ZR, ZR)])

        plsc.subcore_barrier()

        def start_idx(sbuf, dbuf, sem, k):
            pltpu.async_copy(src_hbm.at[pl.ds(base + k, 1)], sbuf, sem)
            pltpu.async_copy(dst_hbm.at[pl.ds(base + k, 1)], dbuf, sem)

        def wait_idx(sbuf, dbuf, sem):
            pltpu.make_async_copy(src_hbm.at[pl.ds(base, 1)], sbuf, sem).wait()
            pltpu.make_async_copy(dst_hbm.at[pl.ds(base, 1)], dbuf, sem).wait()

        def start_gather(sbuf, rbuf, sem):
            pltpu.async_copy(vals_hbm.at[sbuf.at[0]], rbuf, sem)

        def wait_gather(sbuf, rbuf, sem):
            pltpu.make_async_copy(vals_hbm.at[sbuf.at[0]], rbuf, sem).wait()

        start_idx(sidx_a, didx_a, sem_ia, 0)
        start_idx(sidx_b, didx_b, sem_ib, 1)
        wait_idx(sidx_a, didx_a, sem_ia)
        start_gather(sidx_a, rows_a, sem_ga)

        @pl.loop(0, NPAIR)
        def _(j):
            wait_gather(sidx_a, rows_a, sem_ga)
            wait_idx(sidx_b, didx_b, sem_ib)
            start_gather(sidx_b, rows_b, sem_gb)
            pltpu.sync_copy(rows_a, shared.at[didx_a.at[0]], add=True)

            @pl.when(j < NPAIR - 1)
            def _():
                start_idx(sidx_a, didx_a, sem_ia, 2 * j + 2)

            wait_gather(sidx_b, rows_b, sem_gb)

            @pl.when(j < NPAIR - 1)
            def _():
                wait_idx(sidx_a, didx_a, sem_ia)
                start_gather(sidx_a, rows_a, sem_ga)

            pltpu.sync_copy(rows_b, shared.at[didx_b.at[0]], add=True)

            @pl.when(j < NPAIR - 1)
            def _():
                start_idx(sidx_b, didx_b, sem_ib, 2 * j + 3)

        plsc.subcore_barrier()

        @pl.loop(s, NZ, step=NS)
        def _(i):
            pltpu.sync_copy(shared.at[pl.ds(i * ZR, ZR)],
                            acc_hbm.at[c].at[pl.ds(i * ZR, ZR)])

    return _sc_prop


_sc_prop_d = _make_sc_propagate(D)
_sc_prop_do = _make_sc_propagate(DO)


def _tc_mm1_body(x_ref, w_ref, h_ref):
    h_ref[...] = jnp.dot(x_ref[...], w_ref[...],
                         preferred_element_type=jnp.float32)


def _tc_mm1(x, w1):
    return pl.pallas_call(
        _tc_mm1_body,
        grid=(N // RB,),
        in_specs=[
            pl.BlockSpec((RB, D), lambda i: (i, 0)),
            pl.BlockSpec((D, D), lambda i: (0, 0)),
        ],
        out_specs=pl.BlockSpec((RB, D), lambda i: (i, 0)),
        out_shape=jax.ShapeDtypeStruct((N, D), jnp.float32),
    )(x, w1)


def _tc_scale_body(degacc_ref, h1_ref, dinv_ref, h1s_ref):
    deg = degacc_ref[0] + degacc_ref[1] + 1.0
    dinv = lax.rsqrt(deg)
    dinv_ref[...] = dinv
    h1s_ref[...] = h1_ref[...] * dinv[:, :1]


def _tc_scale(degacc, h1):
    return pl.pallas_call(
        _tc_scale_body,
        grid=(N // RB,),
        in_specs=[
            pl.BlockSpec((NC, RB, L), lambda i: (0, i, 0)),
            pl.BlockSpec((RB, D), lambda i: (i, 0)),
        ],
        out_specs=[
            pl.BlockSpec((RB, L), lambda i: (i, 0)),
            pl.BlockSpec((RB, D), lambda i: (i, 0)),
        ],
        out_shape=[
            jax.ShapeDtypeStruct((N, L), jnp.float32),
            jax.ShapeDtypeStruct((N, D), jnp.float32),
        ],
    )(degacc, h1)


def _tc_mid_body(acc_ref, h1_ref, dinv_ref, b1_ref, g_ref, be_ref, w2_ref,
                 h2_ref, h2s_ref):
    dinv = dinv_ref[:, :1]
    out1 = (acc_ref[0] + acc_ref[1]) * dinv + h1_ref[...] * (dinv * dinv)
    out1 = out1 + b1_ref[...]
    act = jnp.maximum(out1, 0.0)
    scale = g_ref[...] * lax.rsqrt(jnp.float32(1.0 + BN_EPS))
    act = act * scale + be_ref[...]
    h2 = jnp.dot(act, w2_ref[...], preferred_element_type=jnp.float32)
    h2_ref[...] = h2
    h2s_ref[...] = h2 * dinv


def _tc_mid(accv, h1, dinv16, b1, gamma, beta, w2p):
    return pl.pallas_call(
        _tc_mid_body,
        grid=(N // RB,),
        in_specs=[
            pl.BlockSpec((NC, RB, D), lambda i: (0, i, 0)),
            pl.BlockSpec((RB, D), lambda i: (i, 0)),
            pl.BlockSpec((RB, L), lambda i: (i, 0)),
            pl.BlockSpec((1, D), lambda i: (0, 0)),
            pl.BlockSpec((1, D), lambda i: (0, 0)),
            pl.BlockSpec((1, D), lambda i: (0, 0)),
            pl.BlockSpec((D, DO), lambda i: (0, 0)),
        ],
        out_specs=[
            pl.BlockSpec((RB, DO), lambda i: (i, 0)),
            pl.BlockSpec((RB, DO), lambda i: (i, 0)),
        ],
        out_shape=[
            jax.ShapeDtypeStruct((N, DO), jnp.float32),
            jax.ShapeDtypeStruct((N, DO), jnp.float32),
        ],
    )(accv, h1, dinv16, b1, gamma, beta, w2p)


def _tc_fin_body(acc_ref, h2_ref, dinv_ref, b2_ref, out_ref):
    dinv = dinv_ref[:, :1]
    out = (acc_ref[0] + acc_ref[1]) * dinv + h2_ref[...] * (dinv * dinv)
    out_ref[...] = out + b2_ref[...]


def _tc_fin(acc2, h2, dinv16, b2p):
    return pl.pallas_call(
        _tc_fin_body,
        grid=(N // RB,),
        in_specs=[
            pl.BlockSpec((NC, RB, DO), lambda i: (0, i, 0)),
            pl.BlockSpec((RB, DO), lambda i: (i, 0)),
            pl.BlockSpec((RB, L), lambda i: (i, 0)),
            pl.BlockSpec((1, DO), lambda i: (0, 0)),
        ],
        out_specs=pl.BlockSpec((RB, DO), lambda i: (i, 0)),
        out_shape=jax.ShapeDtypeStruct((N, DO), jnp.float32),
    )(acc2, h2, dinv16, b2p)


def kernel(x, edge_index, W1, b1, gamma, beta, W2, b2):
    pad = E2 - E
    src = jnp.concatenate(
        [edge_index[0], jnp.zeros((pad,), jnp.int32)]).reshape(NBLK, EB)
    dst = jnp.concatenate(
        [edge_index[1], jnp.full((pad,), N, jnp.int32)]).reshape(NBLK, EB)

    w2p = jnp.zeros((D, DO), jnp.float32).at[:, :3].set(W2)
    b2p = jnp.zeros((1, DO), jnp.float32).at[0, :3].set(b2)

    degacc = _sc_count(dst)
    h1 = _tc_mm1(x, W1)
    dinv16, h1s = _tc_scale(degacc, h1)
    accv = _sc_prop_d(h1s, src, dst)
    h2, h2s = _tc_mid(accv, h1, dinv16,
                      b1.reshape(1, D), gamma.reshape(1, D),
                      beta.reshape(1, D), w2p)
    acc2 = _sc_prop_do(h2s, src, dst)
    out16 = _tc_fin(acc2, h2, dinv16, b2p)
    return out16[:, :3]


# speedup vs baseline: 1.0003x; 1.0003x over previous
import functools

import jax
import jax.numpy as jnp
from jax import lax
from jax.experimental import pallas as pl
from jax.experimental.pallas import tpu as pltpu
from jax.experimental.pallas import tpu_sc as plsc

N = 10000
E = 320000
D = 128
DO = 16
NC = 2
NS = 16
NW = NC * NS
EB = 128
NBW = 80
NBLK = NW * NBW
E2 = NBLK * EB
NPAIR = NBW // 2
NP = 10080
ZR = 80
NZ = NP // ZR
L = 16
BN_EPS = 1e-5
RB = 1000

_mesh = plsc.VectorSubcoreMesh(core_axis_name="c", subcore_axis_name="s")
_sc_params = pltpu.CompilerParams(use_tc_tiling_on_sc=False)


@functools.partial(
    pl.kernel,
    out_type=jax.ShapeDtypeStruct((NC, NP, L), jnp.float32),
    mesh=_mesh,
    compiler_params=_sc_params,
    scratch_types=[
        pltpu.VMEM((EB, L), jnp.float32),
        pltpu.VMEM((1, EB), jnp.int32),
        pltpu.VMEM((1, EB), jnp.int32),
        pltpu.VMEM((ZR, L), jnp.float32),
        pltpu.VMEM_SHARED((NP, L), jnp.float32),
        pltpu.SemaphoreType.DMA,
        pltpu.SemaphoreType.DMA,
    ],
)
def _sc_count(dst_hbm, acc_hbm, ones_v, didx_a, didx_b, zbuf_v, shared,
              sem_a, sem_b):
    c = lax.axis_index("c")
    s = lax.axis_index("s")
    w = s * NC + c
    base = w * NBW

    @pl.loop(0, EB)
    def _(i):
        ones_v[i, :] = jnp.full((L,), 1.0, jnp.float32)

    @pl.loop(0, ZR)
    def _(i):
        zbuf_v[i, :] = jnp.zeros((L,), jnp.float32)

    @pl.loop(s, NZ, step=NS)
    def _(i):
        pltpu.sync_copy(zbuf_v, shared.at[pl.ds(i * ZR, ZR)])

    plsc.subcore_barrier()

    def start_idx(buf, sem, k):
        pltpu.async_copy(dst_hbm.at[pl.ds(base + k, 1)], buf, sem)

    def wait_idx(buf, sem):
        pltpu.make_async_copy(dst_hbm.at[pl.ds(base, 1)], buf, sem).wait()

    start_idx(didx_a, sem_a, 0)
    start_idx(didx_b, sem_b, 1)

    @pl.loop(0, NPAIR)
    def _(j):
        wait_idx(didx_a, sem_a)
        pltpu.sync_copy(ones_v, shared.at[didx_a.at[0]], add=True)

        @pl.when(j < NPAIR - 1)
        def _():
            start_idx(didx_a, sem_a, 2 * j + 2)

        wait_idx(didx_b, sem_b)
        pltpu.sync_copy(ones_v, shared.at[didx_b.at[0]], add=True)

        @pl.when(j < NPAIR - 1)
        def _():
            start_idx(didx_b, sem_b, 2 * j + 3)

    plsc.subcore_barrier()

    @pl.loop(s, NZ, step=NS)
    def _(i):
        pltpu.sync_copy(shared.at[pl.ds(i * ZR, ZR)],
                        acc_hbm.at[c].at[pl.ds(i * ZR, ZR)])


def _make_sc_propagate(width):
    @functools.partial(
        pl.kernel,
        out_type=jax.ShapeDtypeStruct((NC, NP, width), jnp.float32),
        mesh=_mesh,
        compiler_params=_sc_params,
        scratch_types=[
            pltpu.VMEM((1, EB), jnp.int32),
            pltpu.VMEM((1, EB), jnp.int32),
            pltpu.VMEM((1, EB), jnp.int32),
            pltpu.VMEM((1, EB), jnp.int32),
            pltpu.VMEM((EB, width), jnp.float32),
            pltpu.VMEM((EB, width), jnp.float32),
            pltpu.VMEM((ZR, width), jnp.float32),
            pltpu.VMEM_SHARED((NP, width), jnp.float32),
            pltpu.SemaphoreType.DMA,
            pltpu.SemaphoreType.DMA,
            pltpu.SemaphoreType.DMA,
            pltpu.SemaphoreType.DMA,
        ],
    )
    def _sc_prop(vals_hbm, src_hbm, dst_hbm, acc_hbm,
                 sidx_a, didx_a, sidx_b, didx_b, rows_a, rows_b,
                 zbuf_v, shared, sem_ia, sem_ib, sem_ga, sem_gb):
        c = lax.axis_index("c")
        s = lax.axis_index("s")
        w = s * NC + c
        base = w * NBW

        @pl.loop(0, ZR)
        def _(i):
            @pl.loop(0, width, step=L)
            def _(j):
                zbuf_v[i, pl.ds(j, L)] = jnp.zeros((L,), jnp.float32)

        @pl.loop(s, NZ, step=NS)
        def _(i):
            pltpu.sync_copy(zbuf_v, shared.at[pl.ds(i * ZR, ZR)])

        plsc.subcore_barrier()

        def start_idx(sbuf, dbuf, sem, k):
            pltpu.async_copy(src_hbm.at[pl.ds(base + k, 1)], sbuf, sem)
            pltpu.async_copy(dst_hbm.at[pl.ds(base + k, 1)], dbuf, sem)

        def wait_idx(sbuf, dbuf, sem):
            pltpu.make_async_copy(src_hbm.at[pl.ds(base, 1)], sbuf, sem).wait()
            pltpu.make_async_copy(dst_hbm.at[pl.ds(base, 1)], dbuf, sem).wait()

        def start_gather(sbuf, rbuf, sem):
            pltpu.async_copy(vals_hbm.at[sbuf.at[0]], rbuf, sem)

        def wait_gather(sbuf, rbuf, sem):
            pltpu.make_async_copy(vals_hbm.at[sbuf.at[0]], rbuf, sem).wait()

        start_idx(sidx_a, didx_a, sem_ia, 0)
        start_idx(sidx_b, didx_b, sem_ib, 1)
        wait_idx(sidx_a, didx_a, sem_ia)
        start_gather(sidx_a, rows_a, sem_ga)

        @pl.loop(0, NPAIR)
        def _(j):
            wait_gather(sidx_a, rows_a, sem_ga)
            wait_idx(sidx_b, didx_b, sem_ib)
            start_gather(sidx_b, rows_b, sem_gb)
            pltpu.sync_copy(rows_a, shared.at[didx_a.at[0]], add=True)

            @pl.when(j < NPAIR - 1)
            def _():
                start_idx(sidx_a, didx_a, sem_ia, 2 * j + 2)

            wait_gather(sidx_b, rows_b, sem_gb)

            @pl.when(j < NPAIR - 1)
            def _():
                wait_idx(sidx_a, didx_a, sem_ia)
                start_gather(sidx_a, rows_a, sem_ga)

            pltpu.sync_copy(rows_b, shared.at[didx_b.at[0]], add=True)

            @pl.when(j < NPAIR - 1)
            def _():
                start_idx(sidx_b, didx_b, sem_ib, 2 * j + 3)

        plsc.subcore_barrier()

        @pl.loop(s, NZ, step=NS)
        def _(i):
            pltpu.sync_copy(shared.at[pl.ds(i * ZR, ZR)],
                            acc_hbm.at[c].at[pl.ds(i * ZR, ZR)])

    return _sc_prop


_sc_prop_d = _make_sc_propagate(D)
_sc_prop_do = _make_sc_propagate(DO)


def _tc_mm1_body(x_ref, w_ref, h_ref):
    h_ref[...] = jnp.dot(x_ref[...], w_ref[...],
                         preferred_element_type=jnp.float32)


def _tc_mm1(x, w1):
    return pl.pallas_call(
        _tc_mm1_body,
        grid=(N // RB,),
        in_specs=[
            pl.BlockSpec((RB, D), lambda i: (i, 0)),
            pl.BlockSpec((D, D), lambda i: (0, 0)),
        ],
        out_specs=pl.BlockSpec((RB, D), lambda i: (i, 0)),
        out_shape=jax.ShapeDtypeStruct((N, D), jnp.float32),
    )(x, w1)


def _tc_scale_body(degacc_ref, h1_ref, dinv_ref, h1s_ref):
    deg = degacc_ref[0] + degacc_ref[1] + 1.0
    dinv = lax.rsqrt(deg)
    dinv_ref[...] = dinv
    h1s_ref[...] = h1_ref[...] * dinv[:, :1]


def _tc_scale(degacc, h1):
    return pl.pallas_call(
        _tc_scale_body,
        grid=(N // RB,),
        in_specs=[
            pl.BlockSpec((NC, RB, L), lambda i: (0, i, 0)),
            pl.BlockSpec((RB, D), lambda i: (i, 0)),
        ],
        out_specs=[
            pl.BlockSpec((RB, L), lambda i: (i, 0)),
            pl.BlockSpec((RB, D), lambda i: (i, 0)),
        ],
        out_shape=[
            jax.ShapeDtypeStruct((N, L), jnp.float32),
            jax.ShapeDtypeStruct((N, D), jnp.float32),
        ],
    )(degacc, h1)


def _tc_mid_body(acc_ref, h1_ref, dinv_ref, b1_ref, g_ref, be_ref, w2_ref,
                 h2_ref, h2s_ref):
    dinv = dinv_ref[:, :1]
    out1 = (acc_ref[0] + acc_ref[1]) * dinv + h1_ref[...] * (dinv * dinv)
    out1 = out1 + b1_ref[...]
    act = jnp.maximum(out1, 0.0)
    scale = g_ref[...] * lax.rsqrt(jnp.float32(1.0 + BN_EPS))
    act = act * scale + be_ref[...]
    h2 = jnp.dot(act, w2_ref[...], preferred_element_type=jnp.float32)
    h2_ref[...] = h2
    h2s_ref[...] = h2 * dinv


def _tc_mid(accv, h1, dinv16, b1, gamma, beta, w2p):
    return pl.pallas_call(
        _tc_mid_body,
        grid=(N // RB,),
        in_specs=[
            pl.BlockSpec((NC, RB, D), lambda i: (0, i, 0)),
            pl.BlockSpec((RB, D), lambda i: (i, 0)),
            pl.BlockSpec((RB, L), lambda i: (i, 0)),
            pl.BlockSpec((1, D), lambda i: (0, 0)),
            pl.BlockSpec((1, D), lambda i: (0, 0)),
            pl.BlockSpec((1, D), lambda i: (0, 0)),
            pl.BlockSpec((D, DO), lambda i: (0, 0)),
        ],
        out_specs=[
            pl.BlockSpec((RB, DO), lambda i: (i, 0)),
            pl.BlockSpec((RB, DO), lambda i: (i, 0)),
        ],
        out_shape=[
            jax.ShapeDtypeStruct((N, DO), jnp.float32),
            jax.ShapeDtypeStruct((N, DO), jnp.float32),
        ],
    )(accv, h1, dinv16, b1, gamma, beta, w2p)


def _tc_fin_body(acc_ref, h2_ref, dinv_ref, b2_ref, out_ref):
    dinv = dinv_ref[:, :1]
    out = (acc_ref[0] + acc_ref[1]) * dinv + h2_ref[...] * (dinv * dinv)
    out_ref[...] = out + b2_ref[...]


def _tc_fin(acc2, h2, dinv16, b2p):
    return pl.pallas_call(
        _tc_fin_body,
        grid=(N // RB,),
        in_specs=[
            pl.BlockSpec((NC, RB, DO), lambda i: (0, i, 0)),
            pl.BlockSpec((RB, DO), lambda i: (i, 0)),
            pl.BlockSpec((RB, L), lambda i: (i, 0)),
            pl.BlockSpec((1, DO), lambda i: (0, 0)),
        ],
        out_specs=pl.BlockSpec((RB, DO), lambda i: (i, 0)),
        out_shape=jax.ShapeDtypeStruct((N, DO), jnp.float32),
    )(acc2, h2, dinv16, b2p)


def kernel(x, edge_index, W1, b1, gamma, beta, W2, b2):
    pad = E2 - E
    src = jnp.concatenate(
        [edge_index[0], jnp.zeros((pad,), jnp.int32)]).reshape(NBLK, EB)
    trash = N + jnp.arange(pad, dtype=jnp.int32) % (NP - N)
    dst = jnp.concatenate([edge_index[1], trash]).reshape(NBLK, EB)

    w2p = jnp.zeros((D, DO), jnp.float32).at[:, :3].set(W2)
    b2p = jnp.zeros((1, DO), jnp.float32).at[0, :3].set(b2)

    degacc = _sc_count(dst)
    h1 = _tc_mm1(x, W1)
    dinv16, h1s = _tc_scale(degacc, h1)
    accv = _sc_prop_d(h1s, src, dst)
    h2, h2s = _tc_mid(accv, h1, dinv16,
                      b1.reshape(1, D), gamma.reshape(1, D),
                      beta.reshape(1, D), w2p)
    acc2 = _sc_prop_do(h2s, src, dst)
    out16 = _tc_fin(acc2, h2, dinv16, b2p)
    return out16[:, :3]
